# baseline (speedup 1.0000x reference)
import functools

import jax
import jax.numpy as jnp
from jax import lax
from jax.experimental import pallas as pl
from jax.experimental.pallas import tpu as pltpu
from jax.experimental.pallas import tpu_sc as plsc

NC = 2
NS = 16
NW = NC * NS
L = 16

CH = 8
HEAD = 128
TAIL = 72
SPLITS = ((0, HEAD), (HEAD, TAIL))


def _sigmoid(x):
    return 1.0 / (1.0 + jnp.exp(-x))


@functools.partial(jax.jit, static_argnums=(5, 6))
def _run(ids1, table_bf, w_perm, b, didx, batch, hist):
    bags_per_w = batch // NW
    n_chunks = bags_per_w // CH
    rows_per_chunk = CH * hist
    d = table_bf.shape[1]
    nk = d // L
    nh = d // (2 * L)

    mesh = plsc.VectorSubcoreMesh(core_axis_name="c", subcore_axis_name="s")
    assert CH <= L and L % CH == 0
    group = L // CH

    @functools.partial(
        pl.kernel,
        out_type=jax.ShapeDtypeStruct((batch,), jnp.float32),
        mesh=mesh,
        compiler_params=pltpu.CompilerParams(
            needs_layout_passes=False, use_tc_tiling_on_sc=False),
        scratch_types=[
            pltpu.VMEM((3, CH * hist), jnp.int32),
            pltpu.VMEM((2, rows_per_chunk, d), jnp.bfloat16),
            pltpu.VMEM((d,), jnp.float32),
            pltpu.VMEM((L,), jnp.float32),
            pltpu.VMEM((bags_per_w,), jnp.float32),
            pltpu.SemaphoreType.DMA,
            pltpu.SemaphoreType.DMA,
            pltpu.SemaphoreType.DMA,
            pltpu.VMEM_SHARED((NS * CH, 64), jnp.bfloat16),
            pltpu.VMEM((CH, TAIL), jnp.int32),
            pltpu.VMEM((CH, 64), jnp.bfloat16),
            pltpu.VMEM((CH, 64), jnp.bfloat16),
        ],
    )
    def k(ids1_hbm, table_hbm, w_hbm, b_hbm, didx_hbm, out_hbm,
          idx_v, stage_v, w_v, b_v, logit_v, gsem, isem, ssem,
          acc_sh, didx_v, accrd_v, zero_v):
        wid = lax.axis_index("s") * NC + lax.axis_index("c")
        base_bag = wid * bags_per_w

        sid = lax.axis_index("s")
        pltpu.sync_copy(w_hbm, w_v)
        pltpu.sync_copy(b_hbm, b_v)
        pltpu.sync_copy(didx_hbm.at[sid], didx_v)

        zero = jnp.zeros((L,), jnp.float32)
        bzero = jnp.zeros((2 * L,), jnp.bfloat16)
        for s in range(CH):
            for kk in range(nh):
                zero_v[s, pl.ds(kk * 2 * L, 2 * L)] = bzero
        pltpu.sync_copy(zero_v, acc_sh.at[pl.ds(sid * CH, CH)])
        inv_hist = jnp.float32(1.0 / hist)
        lanes = lax.iota(jnp.int32, L)
        wregs = tuple(w_v[pl.ds(kk * L, L)] for kk in range(nk))
        bvec = b_v[pl.ds(0, L)]

        def idx_src(m):
            return ids1_hbm.at[pl.ds((base_bag + m * CH) * hist, CH * hist)]

        def idx_start(m, islot):
            pltpu.async_copy(idx_src(m), idx_v.at[islot], isem)

        def idx_wait(m, islot):
            pltpu.make_async_copy(idx_src(m), idx_v.at[islot], isem).wait()

        def gathers(buf, islot):
            for s in range(CH):
                for off, sz in SPLITS:
                    pltpu.async_copy(
                        table_hbm.at[idx_v.at[islot,
                                              pl.ds(s * hist + off, sz)]],
                        stage_v.at[buf, pl.ds(s * hist + off, sz)], gsem)

        idx_start(0, 0)
        idx_start(1, 1)
        idx_wait(0, 0)
        gathers(0, 0)

        def chunk_body(m, lvec):
            buf = lax.rem(m, 2)
            for s in range(CH):
                for off, sz in SPLITS:
                    pltpu.make_async_copy(
                        table_hbm.at[idx_v.at[lax.rem(m, 3),
                                              pl.ds(s * hist + off, sz)]],
                        stage_v.at[buf, pl.ds(s * hist + off, sz)],
                        gsem).wait()

            @pl.when(m + 1 < n_chunks)
            def _():
                idx_wait(m + 1, lax.rem(m + 1, 3))
                gathers(1 - buf, lax.rem(m + 1, 3))

            @pl.when(m + 2 < n_chunks)
            def _():
                idx_start(m + 2, lax.rem(m + 2, 3))

            for s in range(CH):
                pltpu.async_copy(
                    stage_v.at[buf, pl.ds(s * hist + HEAD, TAIL)],
                    acc_sh.at[didx_v.at[s]], ssem, add=True)

            tec_accs = []
            for s in range(CH):
                def row_body(j, accs):
                    r = s * hist + 4 * j
                    return tuple(
                        accs[p * nh + kk]
                        + stage_v[buf, r + p, pl.ds(kk * 2 * L, 2 * L)]
                        for p in range(4) for kk in range(nh))
                accs = lax.fori_loop(
                    0, HEAD // 4, row_body, (bzero,) * (4 * nh), unroll=2)
                tec_accs.append(accs)

            for s in range(CH):
                pltpu.make_async_copy(
                    stage_v.at[buf, pl.ds(s * hist + HEAD, TAIL)],
                    acc_sh.at[didx_v.at[s]], ssem).wait()
            pltpu.sync_copy(acc_sh.at[pl.ds(sid * CH, CH)], accrd_v)
            pltpu.sync_copy(zero_v, acc_sh.at[pl.ds(sid * CH, CH)])

            for s in range(CH):
                accs = tec_accs[s]
                p16 = zero
                for kk in range(nh):
                    tot = ((accs[0 * nh + kk] + accs[1 * nh + kk])
                           + (accs[2 * nh + kk] + accs[3 * nh + kk]))
                    tot = tot + accrd_v[s, pl.ds(kk * 2 * L, 2 * L)]
                    ae, ao = plsc.unpack(
                        tot, format=plsc.PackFormat.INTERLEAVED,
                        preferred_element_type=jnp.float32)
                    p16 = p16 + _sigmoid(ae * inv_hist) * wregs[2 * kk]
                    p16 = p16 + _sigmoid(ao * inv_hist) * wregs[2 * kk + 1]
                lane = lax.rem(m, group) * CH + s
                lvec = jnp.where(lanes == lane, jnp.sum(p16), lvec)

            @pl.when(lax.rem(m, group) == group - 1)
            def _():
                logit_v[pl.ds((m // group) * L, L)] = lvec

            return lvec

        lax.fori_loop(0, n_chunks, chunk_body, zero)

        for i in range(bags_per_w // L):
            logit_v[pl.ds(i * L, L)] = _sigmoid(
                logit_v[pl.ds(i * L, L)] + bvec)
        pltpu.sync_copy(logit_v, out_hbm.at[pl.ds(base_bag, bags_per_w)])

    return k(ids1, table_bf, w_perm, jnp.broadcast_to(b, (L,)), didx)


def kernel(input_ids, emb_table, W, b):
    batch, hist = input_ids.shape
    assert hist == sum(sz for _, sz in SPLITS) and hist % 8 == 0
    v, d = emb_table.shape
    ids1 = input_ids.reshape(batch * hist)
    table_bf = emb_table.astype(jnp.bfloat16)
    wf = W.reshape(d)
    w_perm = jnp.concatenate(
        [jnp.concatenate([wf[base:base + 2 * L:2],
                          wf[base + 1:base + 2 * L:2]])
         for base in range(0, d, 2 * L)])
    didx = jnp.broadcast_to(
        (jnp.arange(NS, dtype=jnp.int32)[:, None] * CH
         + jnp.arange(CH, dtype=jnp.int32)[None, :])[:, :, None],
        (NS, CH, TAIL))
    out = _run(ids1, table_bf, w_perm, b, didx, batch, hist)
    return out.reshape(batch, 1)


# speedup vs baseline: 1.1536x; 1.1536x over previous
import functools

import jax
import jax.numpy as jnp
from jax import lax
from jax.experimental import pallas as pl
from jax.experimental.pallas import tpu as pltpu
from jax.experimental.pallas import tpu_sc as plsc

NC = 2
NS = 16
NW = NC * NS
L = 16

CH = 8
SPLITS = ((0, 128), (128, 72))


def _sigmoid(x):
    return 1.0 / (1.0 + jnp.exp(-x))


@functools.partial(jax.jit, static_argnums=(4, 5))
def _run(ids1, table_bf, w_perm, b, batch, hist):
    bags_per_w = batch // NW
    n_chunks = bags_per_w // CH
    rows_per_chunk = CH * hist
    d = table_bf.shape[1]
    nk = d // L
    nh = d // (2 * L)

    mesh = plsc.VectorSubcoreMesh(core_axis_name="c", subcore_axis_name="s")
    assert CH <= L and L % CH == 0
    group = L // CH

    @functools.partial(
        pl.kernel,
        out_type=jax.ShapeDtypeStruct((batch,), jnp.float32),
        mesh=mesh,
        compiler_params=pltpu.CompilerParams(
            needs_layout_passes=False, use_tc_tiling_on_sc=False),
        scratch_types=[
            pltpu.VMEM((3, CH * hist), jnp.int32),
            pltpu.VMEM((2, rows_per_chunk, d), jnp.bfloat16),
            pltpu.VMEM((d,), jnp.float32),
            pltpu.VMEM((L,), jnp.float32),
            pltpu.VMEM((bags_per_w,), jnp.float32),
            pltpu.SemaphoreType.DMA,
            pltpu.SemaphoreType.DMA,
        ],
    )
    def k(ids1_hbm, table_hbm, w_hbm, b_hbm, out_hbm,
          idx_v, stage_v, w_v, b_v, logit_v, gsem, isem):
        wid = lax.axis_index("s") * NC + lax.axis_index("c")
        base_bag = wid * bags_per_w

        pltpu.sync_copy(w_hbm, w_v)
        pltpu.sync_copy(b_hbm, b_v)

        zero = jnp.zeros((L,), jnp.float32)
        bzero = jnp.zeros((2 * L,), jnp.bfloat16)
        inv_hist = jnp.float32(1.0 / hist)
        lanes = lax.iota(jnp.int32, L)
        wregs = tuple(w_v[pl.ds(kk * L, L)] for kk in range(nk))
        bvec = b_v[pl.ds(0, L)]

        def idx_src(m):
            return ids1_hbm.at[pl.ds((base_bag + m * CH) * hist, CH * hist)]

        def idx_start(m, islot):
            pltpu.async_copy(idx_src(m), idx_v.at[islot], isem)

        def idx_wait(m, islot):
            pltpu.make_async_copy(idx_src(m), idx_v.at[islot], isem).wait()

        def gathers(buf, islot):
            for s in range(CH):
                for off, sz in SPLITS:
                    pltpu.async_copy(
                        table_hbm.at[idx_v.at[islot,
                                              pl.ds(s * hist + off, sz)]],
                        stage_v.at[buf, pl.ds(s * hist + off, sz)], gsem)

        idx_start(0, 0)
        idx_start(1, 1)
        idx_wait(0, 0)
        gathers(0, 0)

        def chunk_body(m, lvec):
            buf = lax.rem(m, 2)
            for s in range(CH):
                for off, sz in SPLITS:
                    pltpu.make_async_copy(
                        table_hbm.at[idx_v.at[lax.rem(m, 3),
                                              pl.ds(s * hist + off, sz)]],
                        stage_v.at[buf, pl.ds(s * hist + off, sz)],
                        gsem).wait()

            @pl.when(m + 1 < n_chunks)
            def _():
                idx_wait(m + 1, lax.rem(m + 1, 3))
                gathers(1 - buf, lax.rem(m + 1, 3))

            @pl.when(m + 2 < n_chunks)
            def _():
                idx_start(m + 2, lax.rem(m + 2, 3))

            for s in range(CH):
                def row_body(j, accs):
                    r = s * hist + 4 * j
                    return tuple(
                        accs[p * nh + kk]
                        + stage_v[buf, r + p, pl.ds(kk * 2 * L, 2 * L)]
                        for p in range(4) for kk in range(nh))
                accs = lax.fori_loop(
                    0, hist // 4, row_body, (bzero,) * (4 * nh), unroll=4)
                p16 = zero
                for kk in range(nh):
                    tot = ((accs[0 * nh + kk] + accs[1 * nh + kk])
                           + (accs[2 * nh + kk] + accs[3 * nh + kk]))
                    ae, ao = plsc.unpack(
                        tot, format=plsc.PackFormat.INTERLEAVED,
                        preferred_element_type=jnp.float32)
                    p16 = p16 + _sigmoid(ae * inv_hist) * wregs[2 * kk]
                    p16 = p16 + _sigmoid(ao * inv_hist) * wregs[2 * kk + 1]
                lane = lax.rem(m, group) * CH + s
                lvec = jnp.where(lanes == lane, jnp.sum(p16), lvec)

            @pl.when(lax.rem(m, group) == group - 1)
            def _():
                logit_v[pl.ds((m // group) * L, L)] = lvec

            return lvec

        lax.fori_loop(0, n_chunks, chunk_body, zero)

        for i in range(bags_per_w // L):
            logit_v[pl.ds(i * L, L)] = _sigmoid(
                logit_v[pl.ds(i * L, L)] + bvec)
        pltpu.sync_copy(logit_v, out_hbm.at[pl.ds(base_bag, bags_per_w)])

    return k(ids1, table_bf, w_perm, jnp.broadcast_to(b, (L,)))


def kernel(input_ids, emb_table, W, b):
    batch, hist = input_ids.shape
    assert hist == sum(sz for _, sz in SPLITS) and hist % 8 == 0
    v, d = emb_table.shape
    ids1 = input_ids.reshape(batch * hist)
    table_bf = emb_table.astype(jnp.bfloat16)
    wf = W.reshape(d)
    w_perm = jnp.concatenate(
        [jnp.concatenate([wf[base:base + 2 * L:2],
                          wf[base + 1:base + 2 * L:2]])
         for base in range(0, d, 2 * L)])
    out = _run(ids1, table_bf, w_perm, b, batch, hist)
    return out.reshape(batch, 1)


# speedup vs baseline: 1.1994x; 1.0397x over previous
import functools

import jax
import jax.numpy as jnp
from jax import lax
from jax.experimental import pallas as pl
from jax.experimental.pallas import tpu as pltpu
from jax.experimental.pallas import tpu_sc as plsc

NC = 2
NS = 16
NW = NC * NS
L = 16

CH = 8
SPLITS = ((0, 128), (128, 72))


def _sigmoid(x):
    return 1.0 / (1.0 + jnp.exp(-x))


@functools.partial(jax.jit, static_argnums=(4, 5))
def _run(ids1, table_bf, w_perm, b, batch, hist):
    bags_per_w = batch // NW
    n_chunks = bags_per_w // CH
    rows_per_chunk = CH * hist
    d = table_bf.shape[1]
    nk = d // L
    nh = d // (2 * L)

    mesh = plsc.VectorSubcoreMesh(core_axis_name="c", subcore_axis_name="s")
    assert CH <= L and L % CH == 0
    group = L // CH

    @functools.partial(
        pl.kernel,
        out_type=jax.ShapeDtypeStruct((batch,), jnp.float32),
        mesh=mesh,
        compiler_params=pltpu.CompilerParams(
            needs_layout_passes=False, use_tc_tiling_on_sc=False),
        scratch_types=[
            pltpu.VMEM((3, CH * hist), jnp.int32),
            pltpu.VMEM((2, rows_per_chunk, d), jnp.float8_e4m3fn),
            pltpu.VMEM((d,), jnp.float32),
            pltpu.VMEM((L,), jnp.float32),
            pltpu.VMEM((bags_per_w,), jnp.float32),
            pltpu.SemaphoreType.DMA,
            pltpu.SemaphoreType.DMA,
        ],
    )
    def k(ids1_hbm, table_hbm, w_hbm, b_hbm, out_hbm,
          idx_v, stage_v, w_v, b_v, logit_v, gsem, isem):
        wid = lax.axis_index("s") * NC + lax.axis_index("c")
        base_bag = wid * bags_per_w

        pltpu.sync_copy(w_hbm, w_v)
        pltpu.sync_copy(b_hbm, b_v)

        zero = jnp.zeros((L,), jnp.float32)
        bzero = jnp.zeros((2 * L,), jnp.bfloat16)
        inv_hist = jnp.float32(1.0 / hist)
        lanes = lax.iota(jnp.int32, L)
        wregs = tuple(w_v[pl.ds(kk * L, L)] for kk in range(nk))
        bvec = b_v[pl.ds(0, L)]

        def idx_src(m):
            return ids1_hbm.at[pl.ds((base_bag + m * CH) * hist, CH * hist)]

        def idx_start(m, islot):
            pltpu.async_copy(idx_src(m), idx_v.at[islot], isem)

        def idx_wait(m, islot):
            pltpu.make_async_copy(idx_src(m), idx_v.at[islot], isem).wait()

        def gathers(buf, islot):
            for s in range(CH):
                for off, sz in SPLITS:
                    pltpu.async_copy(
                        table_hbm.at[idx_v.at[islot,
                                              pl.ds(s * hist + off, sz)]],
                        stage_v.at[buf, pl.ds(s * hist + off, sz)], gsem)

        idx_start(0, 0)
        idx_start(1, 1)
        idx_wait(0, 0)
        gathers(0, 0)

        def chunk_body(m, lvec):
            buf = lax.rem(m, 2)
            for s in range(CH):
                for off, sz in SPLITS:
                    pltpu.make_async_copy(
                        table_hbm.at[idx_v.at[lax.rem(m, 3),
                                              pl.ds(s * hist + off, sz)]],
                        stage_v.at[buf, pl.ds(s * hist + off, sz)],
                        gsem).wait()

            @pl.when(m + 1 < n_chunks)
            def _():
                idx_wait(m + 1, lax.rem(m + 1, 3))
                gathers(1 - buf, lax.rem(m + 1, 3))

            @pl.when(m + 2 < n_chunks)
            def _():
                idx_start(m + 2, lax.rem(m + 2, 3))

            for s in range(CH):
                def row_body(j, accs):
                    r = s * hist + 4 * j
                    new = []
                    for p in range(4):
                        row = stage_v[buf, r + p, pl.ds(0, 4 * L)]
                        e, o = plsc.unpack(
                            row, format=plsc.PackFormat.INTERLEAVED,
                            preferred_element_type=jnp.bfloat16)
                        new.append(accs[p * 2] + e)
                        new.append(accs[p * 2 + 1] + o)
                    return tuple(new)
                accs = lax.fori_loop(
                    0, hist // 4, row_body, (bzero,) * (4 * nh), unroll=4)
                p16 = zero
                for kk in range(nh):
                    tot = ((accs[0 * nh + kk] + accs[1 * nh + kk])
                           + (accs[2 * nh + kk] + accs[3 * nh + kk]))
                    ae, ao = plsc.unpack(
                        tot, format=plsc.PackFormat.INTERLEAVED,
                        preferred_element_type=jnp.float32)
                    p16 = p16 + _sigmoid(ae * inv_hist) * wregs[2 * kk]
                    p16 = p16 + _sigmoid(ao * inv_hist) * wregs[2 * kk + 1]
                lane = lax.rem(m, group) * CH + s
                lvec = jnp.where(lanes == lane, jnp.sum(p16), lvec)

            @pl.when(lax.rem(m, group) == group - 1)
            def _():
                logit_v[pl.ds((m // group) * L, L)] = lvec

            return lvec

        lax.fori_loop(0, n_chunks, chunk_body, zero)

        for i in range(bags_per_w // L):
            logit_v[pl.ds(i * L, L)] = _sigmoid(
                logit_v[pl.ds(i * L, L)] + bvec)
        pltpu.sync_copy(logit_v, out_hbm.at[pl.ds(base_bag, bags_per_w)])

    return k(ids1, table_bf, w_perm, jnp.broadcast_to(b, (L,)))


def kernel(input_ids, emb_table, W, b):
    batch, hist = input_ids.shape
    assert hist == sum(sz for _, sz in SPLITS) and hist % 8 == 0
    v, d = emb_table.shape
    ids1 = input_ids.reshape(batch * hist)
    table_bf = emb_table.astype(jnp.float8_e4m3fn)
    wf = W.reshape(d)
    w_perm = jnp.concatenate(
        [wf[0::4], wf[2::4], wf[1::4], wf[3::4]])
    out = _run(ids1, table_bf, w_perm, b, batch, hist)
    return out.reshape(batch, 1)


# speedup vs baseline: 1.2800x; 1.0672x over previous
import functools

import jax
import jax.numpy as jnp
from jax import lax
from jax.experimental import pallas as pl
from jax.experimental.pallas import tpu as pltpu
from jax.experimental.pallas import tpu_sc as plsc

NC = 2
NS = 16
NW = NC * NS
L = 16

CH = 16
SPLITS = ((0, 128), (128, 72))


def _sigmoid(x):
    return 1.0 / (1.0 + jnp.exp(-x))


@functools.partial(jax.jit, static_argnums=(4, 5))
def _run(ids1, table_bf, w_perm, b, batch, hist):
    bags_per_w = batch // NW
    n_chunks = bags_per_w // CH
    rows_per_chunk = CH * hist
    d = table_bf.shape[1]
    nk = d // L
    nh = d // (2 * L)

    mesh = plsc.VectorSubcoreMesh(core_axis_name="c", subcore_axis_name="s")
    assert CH <= L and L % CH == 0
    group = L // CH

    @functools.partial(
        pl.kernel,
        out_type=jax.ShapeDtypeStruct((batch,), jnp.float32),
        mesh=mesh,
        compiler_params=pltpu.CompilerParams(
            needs_layout_passes=False, use_tc_tiling_on_sc=False),
        scratch_types=[
            pltpu.VMEM((3, CH * hist), jnp.int32),
            pltpu.VMEM((2, rows_per_chunk, d), jnp.float8_e4m3fn),
            pltpu.VMEM((d,), jnp.float32),
            pltpu.VMEM((L,), jnp.float32),
            pltpu.VMEM((bags_per_w,), jnp.float32),
            pltpu.SemaphoreType.DMA,
            pltpu.SemaphoreType.DMA,
        ],
    )
    def k(ids1_hbm, table_hbm, w_hbm, b_hbm, out_hbm,
          idx_v, stage_v, w_v, b_v, logit_v, gsem, isem):
        wid = lax.axis_index("s") * NC + lax.axis_index("c")
        base_bag = wid * bags_per_w

        pltpu.sync_copy(w_hbm, w_v)
        pltpu.sync_copy(b_hbm, b_v)

        zero = jnp.zeros((L,), jnp.float32)
        bzero = jnp.zeros((2 * L,), jnp.bfloat16)
        inv_hist = jnp.float32(1.0 / hist)
        lanes = lax.iota(jnp.int32, L)
        wregs = tuple(w_v[pl.ds(kk * L, L)] for kk in range(nk))
        bvec = b_v[pl.ds(0, L)]

        def idx_src(m):
            return ids1_hbm.at[pl.ds((base_bag + m * CH) * hist, CH * hist)]

        def idx_start(m, islot):
            pltpu.async_copy(idx_src(m), idx_v.at[islot], isem)

        def idx_wait(m, islot):
            pltpu.make_async_copy(idx_src(m), idx_v.at[islot], isem).wait()

        def gathers(buf, islot):
            for s in range(CH):
                for off, sz in SPLITS:
                    pltpu.async_copy(
                        table_hbm.at[idx_v.at[islot,
                                              pl.ds(s * hist + off, sz)]],
                        stage_v.at[buf, pl.ds(s * hist + off, sz)], gsem)

        idx_start(0, 0)
        idx_start(1, 1)
        idx_wait(0, 0)
        gathers(0, 0)

        def chunk_body(m, lvec):
            buf = lax.rem(m, 2)
            for s in range(CH):
                for off, sz in SPLITS:
                    pltpu.make_async_copy(
                        table_hbm.at[idx_v.at[lax.rem(m, 3),
                                              pl.ds(s * hist + off, sz)]],
                        stage_v.at[buf, pl.ds(s * hist + off, sz)],
                        gsem).wait()

            @pl.when(m + 1 < n_chunks)
            def _():
                idx_wait(m + 1, lax.rem(m + 1, 3))
                gathers(1 - buf, lax.rem(m + 1, 3))

            @pl.when(m + 2 < n_chunks)
            def _():
                idx_start(m + 2, lax.rem(m + 2, 3))

            for s in range(CH):
                def row_body(j, accs):
                    r = s * hist + 4 * j
                    new = []
                    for p in range(4):
                        row = stage_v[buf, r + p, pl.ds(0, 4 * L)]
                        e, o = plsc.unpack(
                            row, format=plsc.PackFormat.INTERLEAVED,
                            preferred_element_type=jnp.bfloat16)
                        new.append(accs[p * 2] + e)
                        new.append(accs[p * 2 + 1] + o)
                    return tuple(new)
                accs = lax.fori_loop(
                    0, hist // 4, row_body, (bzero,) * (4 * nh), unroll=4)
                p16 = zero
                for kk in range(nh):
                    tot = ((accs[0 * nh + kk] + accs[1 * nh + kk])
                           + (accs[2 * nh + kk] + accs[3 * nh + kk]))
                    ae, ao = plsc.unpack(
                        tot, format=plsc.PackFormat.INTERLEAVED,
                        preferred_element_type=jnp.float32)
                    p16 = p16 + _sigmoid(ae * inv_hist) * wregs[2 * kk]
                    p16 = p16 + _sigmoid(ao * inv_hist) * wregs[2 * kk + 1]
                lane = lax.rem(m, group) * CH + s
                lvec = jnp.where(lanes == lane, jnp.sum(p16), lvec)

            @pl.when(lax.rem(m, group) == group - 1)
            def _():
                logit_v[pl.ds((m // group) * L, L)] = lvec

            return lvec

        lax.fori_loop(0, n_chunks, chunk_body, zero)

        for i in range(bags_per_w // L):
            logit_v[pl.ds(i * L, L)] = _sigmoid(
                logit_v[pl.ds(i * L, L)] + bvec)
        pltpu.sync_copy(logit_v, out_hbm.at[pl.ds(base_bag, bags_per_w)])

    return k(ids1, table_bf, w_perm, jnp.broadcast_to(b, (L,)))


def kernel(input_ids, emb_table, W, b):
    batch, hist = input_ids.shape
    assert hist == sum(sz for _, sz in SPLITS) and hist % 8 == 0
    v, d = emb_table.shape
    ids1 = input_ids.reshape(batch * hist)
    table_bf = emb_table.astype(jnp.float8_e4m3fn)
    wf = W.reshape(d)
    w_perm = jnp.concatenate(
        [wf[0::4], wf[2::4], wf[1::4], wf[3::4]])
    out = _run(ids1, table_bf, w_perm, b, batch, hist)
    return out.reshape(batch, 1)
